# Initial kernel scaffold; baseline (speedup 1.0000x reference)
import jax
import jax.numpy as jnp
from jax.experimental import pallas as pl


def kernel(points, normals, scores):
    raise NotImplementedError("write your pallas kernel here")


# speedup vs baseline: 1.3510x; 1.3510x over previous
import functools

import jax
import jax.numpy as jnp
from jax import lax
from jax.experimental import pallas as pl
from jax.experimental.pallas import tpu as pltpu
from jax.experimental.pallas import tpu_sc as plsc

_STRIDE = 4
_GRID = 16
_N = 16384
_M = 4096
_C = 16
_NSEG = _GRID ** 3

_BM = 512
_BN = 2048

_TILES = 16
_PTS_PER_TILE = _M // _TILES
_SEG_PER_TILE = _NSEG // _TILES


def _spt_body(dt_ref, spt_ref):
    d = dt_ref[...]
    mn = jnp.min(d, axis=1, keepdims=True)
    mx = jnp.max(d, axis=1, keepdims=True)
    cell = (d - mn) / (mx - mn + 1e-6)
    ij = jnp.clip((cell * _GRID).astype(jnp.int32), 0, _GRID - 1)
    spt_ref[...] = (ij[0, :] * (_GRID * _GRID) + ij[1, :] * _GRID + ij[2, :])


def _spt_ids(dt):
    return pl.pallas_call(
        _spt_body,
        out_shape=jax.ShapeDtypeStruct((_M,), jnp.int32),
    )(dt)


def _nn_body(a_ref, pt_ref, idx_ref, minv):
    j = pl.program_id(1)

    @pl.when(j == 0)
    def _init():
        minv[...] = jnp.full((_BM,), jnp.inf, jnp.float32)
        idx_ref[...] = jnp.zeros((_BM,), jnp.int32)

    a = a_ref[...]
    pt = pt_ref[...]
    b2 = jnp.sum(pt * pt, axis=0)
    dot = lax.dot_general(a, pt, (((1,), (0,)), ((), ())),
                          preferred_element_type=jnp.float32)
    v = b2[None, :] - 2.0 * dot
    lmin = jnp.min(v, axis=1)
    ji = lax.broadcasted_iota(jnp.int32, (_BM, _BN), 1)
    cand = jnp.where(v == lmin[:, None], ji, _N)
    larg = jnp.min(cand, axis=1) + j * _BN
    upd = lmin < minv[...]
    minv[...] = jnp.where(upd, lmin, minv[...])
    idx_ref[...] = jnp.where(upd, larg, idx_ref[...])


def _nn_argmin(d_points, pt):
    return pl.pallas_call(
        _nn_body,
        grid=(_M // _BM, _N // _BN),
        in_specs=[
            pl.BlockSpec((_BM, 3), lambda i, j: (i, 0)),
            pl.BlockSpec((3, _BN), lambda i, j: (0, j)),
        ],
        out_specs=pl.BlockSpec((_BM,), lambda i, j: (i,)),
        out_shape=jax.ShapeDtypeStruct((_M,), jnp.int32),
        scratch_shapes=[pltpu.VMEM((_BM,), jnp.float32)],
    )(d_points, pt)


def _sc_body(scores_hbm, idx_hbm, spt_hbm, lbl_hbm,
             idx2_v, spt2_v, spt_v, rows_v, zeros_v, ones_v,
             segv, cntv, lblseg_v, lbltab_v, out_v,
             seg_sh, cnt_sh, lbl_sh, sem):
    cid = lax.axis_index("c")
    sid = lax.axis_index("s")

    @pl.when(cid == 0)
    def _body():
        base = sid * _PTS_PER_TILE
        sbase = sid * _SEG_PER_TILE
        for h in range(2):
            pltpu.sync_copy(idx_hbm.at[pl.ds(base + h * 128, 128)], idx2_v.at[h])
            pltpu.sync_copy(spt_hbm.at[pl.ds(base + h * 128, 128)], spt2_v.at[h])
        pltpu.sync_copy(spt_hbm.at[pl.ds(base, _PTS_PER_TILE)], spt_v)

        def _fill(i, carry):
            zeros_v[i] = jnp.zeros((_C,), jnp.float32)
            ones_v[i] = jnp.ones((_C,), jnp.float32)
            return carry
        lax.fori_loop(0, _PTS_PER_TILE, _fill, 0)

        for h in range(2):
            pltpu.async_copy(scores_hbm.at[idx2_v.at[h]],
                             rows_v.at[pl.ds(h * 128, 128)], sem).wait()

        pltpu.sync_copy(zeros_v, seg_sh.at[pl.ds(sbase, _SEG_PER_TILE)])
        pltpu.sync_copy(zeros_v, cnt_sh.at[pl.ds(sbase, _SEG_PER_TILE)])
        plsc.subcore_barrier()

        for h in range(2):
            pltpu.sync_copy(rows_v.at[pl.ds(h * 128, 128)],
                            seg_sh.at[spt2_v.at[h]], add=True)
            pltpu.sync_copy(ones_v.at[pl.ds(h * 128, 128)],
                            cnt_sh.at[spt2_v.at[h]], add=True)
        plsc.subcore_barrier()

        pltpu.sync_copy(seg_sh.at[pl.ds(sbase, _SEG_PER_TILE)], segv)
        pltpu.sync_copy(cnt_sh.at[pl.ds(sbase, _SEG_PER_TILE)], cntv)
        iota16 = lax.iota(jnp.int32, 16)

        def _seg_group(g, carry):
            def _one_seg(j, acc):
                s = g * 16 + j
                mean = segv[s] / jnp.maximum(cntv[s], 1.0)
                m = jnp.max(mean)
                lab = plsc.all_reduce_ffs(mean == m)
                return jnp.where(iota16 == j, lab, acc)
            acc = lax.fori_loop(0, 16, _one_seg, jnp.zeros((16,), jnp.int32))
            lblseg_v[pl.ds(g * 16, 16)] = acc
            return carry
        lax.fori_loop(0, _SEG_PER_TILE // 16, _seg_group, 0)
        pltpu.sync_copy(lblseg_v, lbl_sh.at[pl.ds(sbase, _SEG_PER_TILE)])
        plsc.subcore_barrier()

        pltpu.sync_copy(lbl_sh, lbltab_v)

        def _out_group(g, carry):
            s16 = spt_v[pl.ds(g * 16, 16)]
            out_v[pl.ds(g * 16, 16)] = plsc.load_gather(lbltab_v, [s16])
            return carry
        lax.fori_loop(0, _PTS_PER_TILE // 16, _out_group, 0)
        pltpu.sync_copy(out_v, lbl_hbm.at[pl.ds(base, _PTS_PER_TILE)])


def _sc_labels(scores, indices, sptids):
    mesh = plsc.VectorSubcoreMesh(core_axis_name="c", subcore_axis_name="s")
    kfn = functools.partial(
        pl.kernel,
        mesh=mesh,
        compiler_params=pltpu.CompilerParams(
            needs_layout_passes=False, use_tc_tiling_on_sc=False),
        out_type=jax.ShapeDtypeStruct((_M,), jnp.int32),
        scratch_types=[
            pltpu.VMEM((2, 128), jnp.int32),
            pltpu.VMEM((2, 128), jnp.int32),
            pltpu.VMEM((_PTS_PER_TILE,), jnp.int32),
            pltpu.VMEM((_PTS_PER_TILE, _C), jnp.float32),
            pltpu.VMEM((_PTS_PER_TILE, _C), jnp.float32),
            pltpu.VMEM((_PTS_PER_TILE, _C), jnp.float32),
            pltpu.VMEM((_SEG_PER_TILE, _C), jnp.float32),
            pltpu.VMEM((_SEG_PER_TILE, _C), jnp.float32),
            pltpu.VMEM((_SEG_PER_TILE,), jnp.int32),
            pltpu.VMEM((_NSEG,), jnp.int32),
            pltpu.VMEM((_PTS_PER_TILE,), jnp.int32),
            pltpu.VMEM_SHARED((_NSEG, _C), jnp.float32),
            pltpu.VMEM_SHARED((_NSEG, _C), jnp.float32),
            pltpu.VMEM_SHARED((_NSEG,), jnp.int32),
            pltpu.SemaphoreType.DMA,
        ],
    )(_sc_body)
    return kfn(scores, indices, sptids)


def kernel(points, normals, scores):
    d_points = points[::_STRIDE]
    sptids = _spt_ids(d_points.T)
    indices = _nn_argmin(d_points, points.T)
    labels = _sc_labels(scores, indices, sptids)
    return (d_points, labels[:, None])


# speedup vs baseline: 1.9462x; 1.4406x over previous
import functools

import jax
import jax.numpy as jnp
from jax import lax
from jax.experimental import pallas as pl
from jax.experimental.pallas import tpu as pltpu
from jax.experimental.pallas import tpu_sc as plsc

_STRIDE = 4
_GRID = 16
_N = 16384
_M = 4096
_C = 16
_NSEG = _GRID ** 3

_BM = 1024
_BN = 4096

_TILES = 16
_PTS_PER_TILE = _M // _TILES
_SEG_PER_TILE = _NSEG // _TILES


def _spt_body(dt_ref, spt_ref):
    d = dt_ref[...]
    mn = jnp.min(d, axis=1, keepdims=True)
    mx = jnp.max(d, axis=1, keepdims=True)
    cell = (d - mn) / (mx - mn + 1e-6)
    ij = jnp.clip((cell * _GRID).astype(jnp.int32), 0, _GRID - 1)
    spt_ref[...] = (ij[0, :] * (_GRID * _GRID) + ij[1, :] * _GRID + ij[2, :])


def _spt_ids(dt):
    return pl.pallas_call(
        _spt_body,
        out_shape=jax.ShapeDtypeStruct((_M,), jnp.int32),
    )(dt)


def _nn_body(a_ref, pt_ref, jif_ref, idx_ref, minv):
    j = pl.program_id(1)

    @pl.when(j == 0)
    def _init():
        minv[...] = jnp.full((_BM, 1), jnp.inf, jnp.float32)
        idx_ref[...] = jnp.zeros((_BM, 1), jnp.int32)

    a = a_ref[...]
    pt = pt_ref[...]
    b2 = jnp.sum(pt * pt, axis=0)
    dot = lax.dot_general(a, pt, (((1,), (0,)), ((), ())),
                          preferred_element_type=jnp.float32)
    v = b2[None, :] - 2.0 * dot
    lmin = jnp.min(v, axis=1, keepdims=True)
    cand = jnp.where(v == lmin, jif_ref[...], jnp.float32(2.0 * _N))
    larg = jnp.min(cand, axis=1, keepdims=True).astype(jnp.int32) + j * _BN
    upd = lmin < minv[...]
    minv[...] = jnp.where(upd, lmin, minv[...])
    idx_ref[...] = jnp.where(upd, larg, idx_ref[...])


def _nn_argmin(d_points, pt):
    jif = jnp.arange(_BN, dtype=jnp.float32)[None, :]
    return pl.pallas_call(
        _nn_body,
        grid=(_M // _BM, _N // _BN),
        in_specs=[
            pl.BlockSpec((_BM, 3), lambda i, j: (i, 0)),
            pl.BlockSpec((3, _BN), lambda i, j: (0, j)),
            pl.BlockSpec((1, _BN), lambda i, j: (0, 0)),
        ],
        out_specs=pl.BlockSpec((_BM, 1), lambda i, j: (i, 0)),
        out_shape=jax.ShapeDtypeStruct((_M, 1), jnp.int32),
        scratch_shapes=[pltpu.VMEM((_BM, 1), jnp.float32)],
    )(d_points, pt, jif)


def _sc_body(scores_hbm, idx_hbm, spt_hbm, lbl_hbm,
             idx2_v, spt2_v, spt_v, rows_v, zeros_v, ones_v,
             segv, cntv, lblseg_v, lbltab_v, out_v,
             seg_sh, cnt_sh, lbl_sh, sem):
    cid = lax.axis_index("c")
    sid = lax.axis_index("s")

    @pl.when(cid == 0)
    def _body():
        base = sid * _PTS_PER_TILE
        sbase = sid * _SEG_PER_TILE
        for h in range(2):
            pltpu.sync_copy(idx_hbm.at[pl.ds(base + h * 128, 128)], idx2_v.at[h])
            pltpu.sync_copy(spt_hbm.at[pl.ds(base + h * 128, 128)], spt2_v.at[h])
        pltpu.sync_copy(spt_hbm.at[pl.ds(base, _PTS_PER_TILE)], spt_v)

        def _fill(i, carry):
            zeros_v[i] = jnp.zeros((_C,), jnp.float32)
            ones_v[i] = jnp.ones((_C,), jnp.float32)
            return carry
        lax.fori_loop(0, _PTS_PER_TILE, _fill, 0)

        for h in range(2):
            pltpu.async_copy(scores_hbm.at[idx2_v.at[h]],
                             rows_v.at[pl.ds(h * 128, 128)], sem).wait()

        pltpu.sync_copy(zeros_v, seg_sh.at[pl.ds(sbase, _SEG_PER_TILE)])
        pltpu.sync_copy(zeros_v, cnt_sh.at[pl.ds(sbase, _SEG_PER_TILE)])
        plsc.subcore_barrier()

        for h in range(2):
            pltpu.sync_copy(rows_v.at[pl.ds(h * 128, 128)],
                            seg_sh.at[spt2_v.at[h]], add=True)
            pltpu.sync_copy(ones_v.at[pl.ds(h * 128, 128)],
                            cnt_sh.at[spt2_v.at[h]], add=True)
        plsc.subcore_barrier()

        pltpu.sync_copy(seg_sh.at[pl.ds(sbase, _SEG_PER_TILE)], segv)
        pltpu.sync_copy(cnt_sh.at[pl.ds(sbase, _SEG_PER_TILE)], cntv)
        iota16 = lax.iota(jnp.int32, 16)

        def _seg_group(g, carry):
            def _one_seg(j, acc):
                s = g * 16 + j
                mean = segv[s] / jnp.maximum(cntv[s], 1.0)
                m = jnp.max(mean)
                lab = plsc.all_reduce_ffs(mean == m)
                return jnp.where(iota16 == j, lab, acc)
            acc = lax.fori_loop(0, 16, _one_seg, jnp.zeros((16,), jnp.int32))
            lblseg_v[pl.ds(g * 16, 16)] = acc
            return carry
        lax.fori_loop(0, _SEG_PER_TILE // 16, _seg_group, 0)
        pltpu.sync_copy(lblseg_v, lbl_sh.at[pl.ds(sbase, _SEG_PER_TILE)])
        plsc.subcore_barrier()

        pltpu.sync_copy(lbl_sh, lbltab_v)

        def _out_group(g, carry):
            s16 = spt_v[pl.ds(g * 16, 16)]
            out_v[pl.ds(g * 16, 16)] = plsc.load_gather(lbltab_v, [s16])
            return carry
        lax.fori_loop(0, _PTS_PER_TILE // 16, _out_group, 0)
        pltpu.sync_copy(out_v, lbl_hbm.at[pl.ds(base, _PTS_PER_TILE)])


def _sc_labels(scores, indices, sptids):
    mesh = plsc.VectorSubcoreMesh(core_axis_name="c", subcore_axis_name="s")
    kfn = functools.partial(
        pl.kernel,
        mesh=mesh,
        compiler_params=pltpu.CompilerParams(
            needs_layout_passes=False, use_tc_tiling_on_sc=False),
        out_type=jax.ShapeDtypeStruct((_M,), jnp.int32),
        scratch_types=[
            pltpu.VMEM((2, 128), jnp.int32),
            pltpu.VMEM((2, 128), jnp.int32),
            pltpu.VMEM((_PTS_PER_TILE,), jnp.int32),
            pltpu.VMEM((_PTS_PER_TILE, _C), jnp.float32),
            pltpu.VMEM((_PTS_PER_TILE, _C), jnp.float32),
            pltpu.VMEM((_PTS_PER_TILE, _C), jnp.float32),
            pltpu.VMEM((_SEG_PER_TILE, _C), jnp.float32),
            pltpu.VMEM((_SEG_PER_TILE, _C), jnp.float32),
            pltpu.VMEM((_SEG_PER_TILE,), jnp.int32),
            pltpu.VMEM((_NSEG,), jnp.int32),
            pltpu.VMEM((_PTS_PER_TILE,), jnp.int32),
            pltpu.VMEM_SHARED((_NSEG, _C), jnp.float32),
            pltpu.VMEM_SHARED((_NSEG, _C), jnp.float32),
            pltpu.VMEM_SHARED((_NSEG,), jnp.int32),
            pltpu.SemaphoreType.DMA,
        ],
    )(_sc_body)
    return kfn(scores, indices, sptids)


def kernel(points, normals, scores):
    d_points = points[::_STRIDE]
    sptids = _spt_ids(d_points.T)
    indices = _nn_argmin(d_points, points.T)[:, 0]
    labels = _sc_labels(scores, indices, sptids)
    return (d_points, labels[:, None])
